# Initial kernel scaffold; baseline (speedup 1.0000x reference)
import jax
import jax.numpy as jnp
from jax.experimental import pallas as pl


def kernel(x, edge_index, bn0_gamma, bn0_beta, W1, b1, bn1_gamma, bn1_beta, W2, b2):
    raise NotImplementedError("write your pallas kernel here")


# speedup vs baseline: 7.3568x; 7.3568x over previous
import functools

import jax
import jax.numpy as jnp
from jax import lax
from jax.experimental import pallas as pl
from jax.experimental.pallas import tpu as pltpu
from jax.experimental.pallas import tpu_sc as plsc

N = 10000
E = 320000
D = 128
H = 128
C = 40
CP = 48

NC = 2
NS = 16
NW = NC * NS
EP = E // NW
KC = 100
NCH = EP // KC

ROWS_A = 632
ROWS_B = N - 15 * ROWS_A


def _seg_sum_sc(width):
  mesh = plsc.VectorSubcoreMesh(core_axis_name="c", subcore_axis_name="s")

  @functools.partial(
      pl.kernel,
      out_type=jax.ShapeDtypeStruct((NC, N, width), jnp.float32),
      mesh=mesh,
      compiler_params=pltpu.CompilerParams(use_tc_tiling_on_sc=False),
      scratch_types=[
          pltpu.VMEM((2, 2, KC), jnp.int32),
          pltpu.VMEM((2, KC, width), jnp.float32),
          pltpu.VMEM_SHARED((N, width), jnp.float32),
          pltpu.SemaphoreType.DMA,
          pltpu.SemaphoreType.DMA,
      ],
  )
  def seg_sum(h_hbm, idx_hbm, zeros_hbm, out_hbm,
              idx_v, rows_v, acc_sh, sem0, sem1):
    cid = lax.axis_index("c")
    sid = lax.axis_index("s")
    wid = sid * NC + cid

    @pl.when(sid < NS - 1)
    def _():
      pltpu.sync_copy(zeros_hbm.at[pl.ds(sid * ROWS_A, ROWS_A)],
                      acc_sh.at[pl.ds(sid * ROWS_A, ROWS_A)])

    @pl.when(sid == NS - 1)
    def _():
      pltpu.sync_copy(zeros_hbm.at[pl.ds((NS - 1) * ROWS_A, ROWS_B)],
                      acc_sh.at[pl.ds((NS - 1) * ROWS_A, ROWS_B)])

    plsc.subcore_barrier()

    def body(g, _):
      c0 = 2 * g
      c1 = c0 + 1
      pltpu.sync_copy(idx_hbm.at[wid, c0], idx_v.at[0])
      pltpu.sync_copy(idx_hbm.at[wid, c1], idx_v.at[1])
      cp0 = pltpu.async_copy(h_hbm.at[idx_v.at[0, 0]], rows_v.at[0], sem0)
      cp1 = pltpu.async_copy(h_hbm.at[idx_v.at[1, 0]], rows_v.at[1], sem1)
      cp0.wait()
      pltpu.sync_copy(rows_v.at[0], acc_sh.at[idx_v.at[0, 1]], add=True)
      cp1.wait()
      pltpu.sync_copy(rows_v.at[1], acc_sh.at[idx_v.at[1, 1]], add=True)
      return 0

    lax.fori_loop(0, NCH // 2, body, 0)

    plsc.subcore_barrier()

    @pl.when(sid < NS - 1)
    def _():
      pltpu.sync_copy(acc_sh.at[pl.ds(sid * ROWS_A, ROWS_A)],
                      out_hbm.at[cid, pl.ds(sid * ROWS_A, ROWS_A)])

    @pl.when(sid == NS - 1)
    def _():
      pltpu.sync_copy(acc_sh.at[pl.ds((NS - 1) * ROWS_A, ROWS_B)],
                      out_hbm.at[cid, pl.ds((NS - 1) * ROWS_A, ROWS_B)])

  return seg_sum


_seg128 = _seg_sum_sc(D)
_seg48 = _seg_sum_sc(CP)


def _bn0_body(x_ref, g_ref, b_ref, o_ref):
  x = x_ref[...]
  mu = jnp.mean(x, axis=0, keepdims=True)
  var = jnp.mean((x - mu) * (x - mu), axis=0, keepdims=True)
  o_ref[...] = (x - mu) * lax.rsqrt(var + 1e-5) * g_ref[...] + b_ref[...]


def _mid_body(h0_ref, agg_ref, w1_ref, b1_ref, g_ref, be_ref, w2_ref, z_ref):
  t = h0_ref[...] + agg_ref[0] + agg_ref[1]
  h1 = jnp.dot(t, w1_ref[...], preferred_element_type=jnp.float32) + b1_ref[...]
  mu = jnp.mean(h1, axis=0, keepdims=True)
  var = jnp.mean((h1 - mu) * (h1 - mu), axis=0, keepdims=True)
  h2 = jnp.maximum(
      (h1 - mu) * lax.rsqrt(var + 1e-5) * g_ref[...] + be_ref[...], 0.0)
  z_ref[...] = jnp.dot(h2, w2_ref[...], preferred_element_type=jnp.float32)


def _fin_body(z_ref, agg_ref, b2_ref, o_ref):
  o_ref[...] = z_ref[...] + agg_ref[0] + agg_ref[1] + b2_ref[...]


def kernel(x, edge_index, bn0_gamma, bn0_beta, W1, b1, bn1_gamma, bn1_beta,
           W2, b2):
  src = edge_index[0].astype(jnp.int32).reshape(NW, NCH, KC)
  dst = edge_index[1].astype(jnp.int32).reshape(NW, NCH, KC)
  idx = jnp.stack([src, dst], axis=2)

  h0 = pl.pallas_call(
      _bn0_body,
      out_shape=jax.ShapeDtypeStruct((N, D), jnp.float32),
  )(x, bn0_gamma.reshape(1, D), bn0_beta.reshape(1, D))

  zeros128 = jnp.zeros((N, D), jnp.float32)
  agg1 = _seg128(h0, idx, zeros128)

  w2p = jnp.pad(W2, ((0, 0), (0, CP - C)))
  z = pl.pallas_call(
      _mid_body,
      out_shape=jax.ShapeDtypeStruct((N, CP), jnp.float32),
  )(h0, agg1, W1, b1.reshape(1, H), bn1_gamma.reshape(1, H),
    bn1_beta.reshape(1, H), w2p)

  zeros48 = jnp.zeros((N, CP), jnp.float32)
  agg2 = _seg48(z, idx, zeros48)

  b2p = jnp.pad(b2, (0, CP - C)).reshape(1, CP)
  outp = pl.pallas_call(
      _fin_body,
      out_shape=jax.ShapeDtypeStruct((N, CP), jnp.float32),
  )(z, agg2, b2p)
  return outp[:, :C]


# speedup vs baseline: 10.0887x; 1.3713x over previous
import functools

import jax
import jax.numpy as jnp
from jax import lax
from jax.experimental import pallas as pl
from jax.experimental.pallas import tpu as pltpu
from jax.experimental.pallas import tpu_sc as plsc

N = 10000
E = 320000
D = 128
H = 128
C = 40
CP = 48

NC = 2
NS = 16
NW = NC * NS
EP = E // NW
KC = 50
NCH = EP // KC
NBUF = 4
GB = 100
NBLK = NCH // GB

ROWS_A = 632
ROWS_B = N - 15 * ROWS_A


def _seg_sum_sc(width):
  mesh = plsc.VectorSubcoreMesh(core_axis_name="c", subcore_axis_name="s")

  @functools.partial(
      pl.kernel,
      out_type=jax.ShapeDtypeStruct((NC, N, width), jnp.float32),
      mesh=mesh,
      compiler_params=pltpu.CompilerParams(use_tc_tiling_on_sc=False),
      scratch_types=[
          pltpu.VMEM((GB, 2, KC), jnp.int32),
          pltpu.VMEM((NBUF, KC, width), jnp.float32),
          pltpu.VMEM_SHARED((N, width), jnp.float32),
          [pltpu.SemaphoreType.DMA] * NBUF,
          [pltpu.SemaphoreType.DMA] * NBUF,
      ],
  )
  def seg_sum(h_hbm, idx_hbm, zeros_hbm, out_hbm,
              idx_v, rows_v, acc_sh, gsems, ssems):
    cid = lax.axis_index("c")
    sid = lax.axis_index("s")
    wid = sid * NC + cid

    def gather(ch, b):
      return pltpu.async_copy(h_hbm.at[idx_v.at[ch, 0]], rows_v.at[b],
                              gsems[b])

    def scatter(ch, b):
      return pltpu.async_copy(rows_v.at[b], acc_sh.at[idx_v.at[ch, 1]],
                              ssems[b], add=True)

    @pl.when(sid < NS - 1)
    def _():
      pltpu.sync_copy(zeros_hbm.at[pl.ds(sid * ROWS_A, ROWS_A)],
                      acc_sh.at[pl.ds(sid * ROWS_A, ROWS_A)])

    @pl.when(sid == NS - 1)
    def _():
      pltpu.sync_copy(zeros_hbm.at[pl.ds((NS - 1) * ROWS_A, ROWS_B)],
                      acc_sh.at[pl.ds((NS - 1) * ROWS_A, ROWS_B)])

    plsc.subcore_barrier()

    for blk in range(NBLK):
      pltpu.sync_copy(idx_hbm.at[wid, pl.ds(blk * GB, GB)], idx_v)
      for b in range(NBUF):
        gather(b, b)

      def body(g, _):
        base = NBUF * g
        for b in range(NBUF):
          pltpu.make_async_copy(h_hbm.at[idx_v.at[base + b, 0]],
                                rows_v.at[b], gsems[b]).wait()
          scatter(base + b, b)
        for b in range(NBUF):
          pltpu.make_async_copy(rows_v.at[b],
                                acc_sh.at[idx_v.at[base + b, 1]],
                                ssems[b]).wait()
          gather(base + NBUF + b, b)
        return 0

      lax.fori_loop(0, GB // NBUF - 1, body, 0)

      last = GB - NBUF
      for b in range(NBUF):
        pltpu.make_async_copy(h_hbm.at[idx_v.at[last + b, 0]],
                              rows_v.at[b], gsems[b]).wait()
        scatter(last + b, b)
      for b in range(NBUF):
        pltpu.make_async_copy(rows_v.at[b],
                              acc_sh.at[idx_v.at[last + b, 1]],
                              ssems[b]).wait()

    plsc.subcore_barrier()

    @pl.when(sid < NS - 1)
    def _():
      pltpu.sync_copy(acc_sh.at[pl.ds(sid * ROWS_A, ROWS_A)],
                      out_hbm.at[cid, pl.ds(sid * ROWS_A, ROWS_A)])

    @pl.when(sid == NS - 1)
    def _():
      pltpu.sync_copy(acc_sh.at[pl.ds((NS - 1) * ROWS_A, ROWS_B)],
                      out_hbm.at[cid, pl.ds((NS - 1) * ROWS_A, ROWS_B)])

  return seg_sum


_seg128 = _seg_sum_sc(D)
_seg48 = _seg_sum_sc(CP)


def _bn0_body(x_ref, g_ref, b_ref, o_ref):
  x = x_ref[...]
  mu = jnp.mean(x, axis=0, keepdims=True)
  var = jnp.mean((x - mu) * (x - mu), axis=0, keepdims=True)
  o_ref[...] = (x - mu) * lax.rsqrt(var + 1e-5) * g_ref[...] + b_ref[...]


def _mid_body(h0_ref, agg_ref, w1_ref, b1_ref, g_ref, be_ref, w2_ref, z_ref):
  t = h0_ref[...] + agg_ref[0] + agg_ref[1]
  h1 = jnp.dot(t, w1_ref[...], preferred_element_type=jnp.float32) + b1_ref[...]
  mu = jnp.mean(h1, axis=0, keepdims=True)
  var = jnp.mean((h1 - mu) * (h1 - mu), axis=0, keepdims=True)
  h2 = jnp.maximum(
      (h1 - mu) * lax.rsqrt(var + 1e-5) * g_ref[...] + be_ref[...], 0.0)
  z_ref[...] = jnp.dot(h2, w2_ref[...], preferred_element_type=jnp.float32)


def _fin_body(z_ref, agg_ref, b2_ref, o_ref):
  o_ref[...] = z_ref[...] + agg_ref[0] + agg_ref[1] + b2_ref[...]


def kernel(x, edge_index, bn0_gamma, bn0_beta, W1, b1, bn1_gamma, bn1_beta,
           W2, b2):
  src = edge_index[0].astype(jnp.int32).reshape(NW, NCH, KC)
  dst = edge_index[1].astype(jnp.int32).reshape(NW, NCH, KC)
  idx = jnp.stack([src, dst], axis=2)

  h0 = pl.pallas_call(
      _bn0_body,
      out_shape=jax.ShapeDtypeStruct((N, D), jnp.float32),
  )(x, bn0_gamma.reshape(1, D), bn0_beta.reshape(1, D))

  zeros128 = jnp.zeros((N, D), jnp.float32)
  agg1 = _seg128(h0, idx, zeros128)

  w2p = jnp.pad(W2, ((0, 0), (0, CP - C)))
  z = pl.pallas_call(
      _mid_body,
      out_shape=jax.ShapeDtypeStruct((N, CP), jnp.float32),
  )(h0, agg1, W1, b1.reshape(1, H), bn1_gamma.reshape(1, H),
    bn1_beta.reshape(1, H), w2p)

  zeros48 = jnp.zeros((N, CP), jnp.float32)
  agg2 = _seg48(z, idx, zeros48)

  b2p = jnp.pad(b2, (0, CP - C)).reshape(1, CP)
  outp = pl.pallas_call(
      _fin_body,
      out_shape=jax.ShapeDtypeStruct((N, CP), jnp.float32),
  )(z, agg2, b2p)
  return outp[:, :C]
